# baseline (speedup 1.0000x reference)
import functools

import jax
import jax.numpy as jnp
from jax import lax
from jax.experimental import pallas as pl
from jax.experimental.pallas import tpu as pltpu
from jax.experimental.pallas import tpu_sc as plsc

B = 16384
D = 64
S = 20
NW = 32
BPW = B // NW
CB = 32
NCHUNK = BPW // CB
GPC = CB * S // 128


@functools.partial(
    pl.kernel,
    mesh=plsc.VectorSubcoreMesh(core_axis_name="c", subcore_axis_name="s"),
    compiler_params=pltpu.CompilerParams(use_tc_tiling_on_sc=False),
    out_type=jax.ShapeDtypeStruct((B, D), jnp.float32),
    scratch_types=[
        pltpu.VMEM((4, 128), jnp.int32),
        pltpu.VMEM((NCHUNK * GPC, 128), jnp.int32),
        pltpu.VMEM((BPW, D), jnp.float32),
        pltpu.VMEM((CB * S, D), jnp.float32),
        pltpu.VMEM((CB * S, D), jnp.float32),
        pltpu.SemaphoreType.DMA,
        pltpu.SemaphoreType.DMA,
        pltpu.SemaphoreType.DMA,
    ],
)
def _fasttext_sc(widx_hbm, sidx_hbm, wemb_hbm, semb_hbm, out_hbm,
                 widx_v, sidx_v, outbuf, gbuf0, gbuf1, wsem, sem0, sem1):
    wid = lax.axis_index("s") * 2 + lax.axis_index("c")
    base = wid * BPW

    pltpu.sync_copy(widx_hbm.at[pl.ds(wid * 4, 4), :], widx_v)
    pltpu.sync_copy(sidx_hbm.at[pl.ds(wid * (NCHUNK * GPC), NCHUNK * GPC), :],
                    sidx_v)

    whs = [
        pltpu.async_copy(wemb_hbm.at[widx_v.at[q]],
                         outbuf.at[pl.ds(q * 128, 128), :], wsem)
        for q in range(4)
    ]

    gbufs = [gbuf0, gbuf1]
    sems = [sem0, sem1]
    for half in range(2):
        for i in range(GPC):
            pltpu.async_copy(semb_hbm.at[sidx_v.at[half * GPC + i]],
                             gbufs[half].at[pl.ds(i * 128, 128), :],
                             sems[half])
    for h in whs:
        h.wait()

    def outer(p, carry):
        for half in range(2):
            c = p * 2 + half
            gb = gbufs[half]
            sem = sems[half]
            for i in range(GPC):
                pltpu.make_async_copy(semb_hbm.at[pl.ds(0, 128), :],
                                      gb.at[pl.ds(i * 128, 128), :],
                                      sem).wait()

            def body(b, inner_carry, c=c, gb=gb):
                row = c * CB + b
                for k in range(D // 16):
                    acc = outbuf[row, pl.ds(k * 16, 16)] * jnp.float32(S)
                    for s in range(S):
                        acc = acc + gb[b * S + s, pl.ds(k * 16, 16)]
                    outbuf[row, pl.ds(k * 16, 16)] = acc * jnp.float32(0.5 / S)
                return inner_carry

            lax.fori_loop(0, CB, body, 0)

            @pl.when(c + 2 < NCHUNK)
            def _(c=c, gb=gb, sem=sem):
                for i in range(GPC):
                    pltpu.async_copy(semb_hbm.at[sidx_v.at[(c + 2) * GPC + i]],
                                     gb.at[pl.ds(i * 128, 128), :], sem)
        return carry

    lax.fori_loop(0, NCHUNK // 2, outer, 0)

    pltpu.sync_copy(outbuf, out_hbm.at[pl.ds(base, BPW), :])


def kernel(word_idx, subword_idx, word_emb, subword_emb):
    widx = word_idx.astype(jnp.int32).reshape(B // 128, 128)
    sidx = subword_idx.astype(jnp.int32).reshape(B * S // 128, 128)
    return _fasttext_sc(widx, sidx, word_emb, subword_emb)
